# baseline (speedup 1.0000x reference)
import functools

import jax
import jax.numpy as jnp
from jax import lax
from jax.experimental import pallas as pl
from jax.experimental.pallas import tpu as pltpu
from jax.experimental.pallas import tpu_sc as plsc

_MAX_LEN = 200
_EMBED_DIM = 64
_FLAT = _MAX_LEN * _EMBED_DIM
_BATCH = 4096
_NC = 2
_NS = 16
_NW = _NC * _NS
_B_PER_W = _BATCH // _NW
_BLK = 8
_N_COPIES = _B_PER_W // _BLK


def _sc_body(pe_hbm, out_hbm, buf, sem):
    wid = lax.axis_index("s") * _NC + lax.axis_index("c")
    base = wid * _B_PER_W
    for r in range(_BLK):
        pltpu.sync_copy(pe_hbm, buf.at[pl.ds(r, 1)])
    copies = [
        pltpu.make_async_copy(
            buf, out_hbm.at[pl.ds(base + j * _BLK, _BLK), :], sem
        )
        for j in range(_N_COPIES)
    ]
    for c in copies:
        c.start()
    for c in copies:
        c.wait()


_sc_broadcast = functools.partial(
    pl.kernel,
    out_type=jax.ShapeDtypeStruct((_BATCH, _FLAT), jnp.float32),
    mesh=plsc.VectorSubcoreMesh(core_axis_name="c", subcore_axis_name="s"),
    scratch_types=[
        pltpu.VMEM((_BLK, _FLAT), jnp.float32),
        pltpu.SemaphoreType.DMA,
    ],
)(_sc_body)


def kernel(x, pe_weight):
    batch = x.shape[0]
    pe_flat = pe_weight.reshape(1, _FLAT)
    out = _sc_broadcast(pe_flat)
    return out.reshape(batch, _MAX_LEN, _EMBED_DIM)
